# baseline (speedup 1.0000x reference)
import functools

import jax
import jax.numpy as jnp
from jax import lax
from jax.experimental import pallas as pl
from jax.experimental.pallas import tpu as pltpu
from jax.experimental.pallas import tpu_sc as plsc

VOCAB = 100000
DIM = 128
BATCH = 16384
NEG = 20

_info = plsc.get_sparse_core_info()
NC = _info.num_cores
NS = _info.num_subcores
NW = NC * NS

CH = 128
W_CHUNKS = BATCH // (NW * CH)
N_CHUNKS = BATCH * NEG // (NW * CH)
W_PER = W_CHUNKS * CH
N_PER = N_CHUNKS * CH

_mesh = plsc.VectorSubcoreMesh(core_axis_name="c", subcore_axis_name="s")


@functools.partial(
    pl.kernel,
    mesh=_mesh,
    out_type=[
        jax.ShapeDtypeStruct((BATCH, DIM), jnp.float32),
        jax.ShapeDtypeStruct((BATCH, DIM), jnp.float32),
        jax.ShapeDtypeStruct((BATCH * NEG, DIM), jnp.float32),
    ],
    scratch_types=[
        pltpu.VMEM((W_CHUNKS, CH), jnp.int32),
        pltpu.VMEM((W_CHUNKS, CH), jnp.int32),
        pltpu.VMEM((N_CHUNKS, CH), jnp.int32),
        pltpu.VMEM((2 * CH, DIM), jnp.float32),
        pltpu.VMEM((2 * CH, DIM), jnp.float32),
        pltpu.SemaphoreType.DMA,
        pltpu.SemaphoreType.DMA,
        pltpu.SemaphoreType.DMA,
        pltpu.SemaphoreType.DMA,
    ],
)
def _sc_gather(words_hbm, ctx_hbm, neg_hbm, wtab_hbm, ctab_hbm,
               out_w, out_c, out_n,
               idx_w, idx_c, idx_n, buf_a, buf_b,
               gsem_a, gsem_b, ssem_a, ssem_b):
    wid = lax.axis_index("s") * NC + lax.axis_index("c")

    pltpu.sync_copy(words_hbm.at[wid], idx_w)
    pltpu.sync_copy(ctx_hbm.at[wid], idx_c)
    pltpu.sync_copy(neg_hbm.at[wid], idx_n)

    def phase(tab, idx_v, out, base, nchunks):
        def g_start(c, buf, sem):
            for h in range(2):
                pltpu.async_copy(tab.at[idx_v.at[2 * c + h]],
                                 buf.at[pl.ds(h * CH, CH)], sem)

        def g_wait(c, buf, sem):
            for h in range(2):
                pltpu.make_async_copy(tab.at[idx_v.at[2 * c + h]],
                                      buf.at[pl.ds(h * CH, CH)], sem).wait()

        def s_start(c, buf, sem):
            pltpu.async_copy(buf, out.at[pl.ds(base + c * 2 * CH, 2 * CH)],
                             sem)

        def s_wait(c, buf, sem):
            pltpu.make_async_copy(
                buf, out.at[pl.ds(base + c * 2 * CH, 2 * CH)], sem).wait()

        g_start(0, buf_a, gsem_a)
        g_start(1, buf_b, gsem_b)

        def body(m, carry):
            c = m * 2
            g_wait(c, buf_a, gsem_a)
            s_start(c, buf_a, ssem_a)
            g_wait(c + 1, buf_b, gsem_b)
            s_start(c + 1, buf_b, ssem_b)
            s_wait(c, buf_a, ssem_a)
            g_start(c + 2, buf_a, gsem_a)
            s_wait(c + 1, buf_b, ssem_b)
            g_start(c + 3, buf_b, gsem_b)
            return carry
        lax.fori_loop(0, nchunks // 2 - 1, body, 0)

        c = nchunks - 2
        g_wait(c, buf_a, gsem_a)
        s_start(c, buf_a, ssem_a)
        g_wait(c + 1, buf_b, gsem_b)
        s_start(c + 1, buf_b, ssem_b)
        s_wait(c, buf_a, ssem_a)
        s_wait(c + 1, buf_b, ssem_b)

    phase(wtab_hbm, idx_w, out_w, wid * W_PER, W_CHUNKS // 2)
    phase(ctab_hbm, idx_c, out_c, wid * W_PER, W_CHUNKS // 2)
    phase(ctab_hbm, idx_n, out_n, wid * N_PER, N_CHUNKS // 2)


def kernel(words, contexts, neg_contexts, w_embeddings, c_embeddings):
    words3 = words.astype(jnp.int32).reshape(NW, W_CHUNKS, CH)
    ctx3 = contexts.astype(jnp.int32).reshape(NW, W_CHUNKS, CH)
    neg3 = neg_contexts.astype(jnp.int32).T.reshape(NW, N_CHUNKS, CH)
    out_w, out_c, out_nf = _sc_gather(words3, ctx3, neg3,
                                      w_embeddings, c_embeddings)
    out_n = out_nf.reshape(NEG, BATCH, DIM).transpose(1, 0, 2)
    return (out_w, out_c, out_n)


# speedup vs baseline: 1.0395x; 1.0395x over previous
import functools

import jax
import jax.numpy as jnp
from jax import lax
from jax.experimental import pallas as pl
from jax.experimental.pallas import tpu as pltpu
from jax.experimental.pallas import tpu_sc as plsc

VOCAB = 100000
DIM = 128
BATCH = 16384
NEG = 20

_info = plsc.get_sparse_core_info()
NC = _info.num_cores
NS = _info.num_subcores
NW = NC * NS

CH = 128
W_CHUNKS = BATCH // (NW * CH)
N_CHUNKS = BATCH * NEG // (NW * CH)
W_PER = W_CHUNKS * CH
N_PER = N_CHUNKS * CH

_mesh = plsc.VectorSubcoreMesh(core_axis_name="c", subcore_axis_name="s")


@functools.partial(
    pl.kernel,
    mesh=_mesh,
    out_type=[
        jax.ShapeDtypeStruct((BATCH, DIM), jnp.float32),
        jax.ShapeDtypeStruct((BATCH, DIM), jnp.float32),
        jax.ShapeDtypeStruct((BATCH * NEG, DIM), jnp.float32),
    ],
    scratch_types=[
        pltpu.VMEM((W_CHUNKS, CH), jnp.int32),
        pltpu.VMEM((W_CHUNKS, CH), jnp.int32),
        pltpu.VMEM((N_CHUNKS, CH), jnp.int32),
        pltpu.VMEM((CH, DIM), jnp.float32),
        pltpu.VMEM((CH, DIM), jnp.float32),
        pltpu.VMEM((CH, DIM), jnp.float32),
        pltpu.VMEM((CH, DIM), jnp.float32),
        pltpu.SemaphoreType.DMA,
        pltpu.SemaphoreType.DMA,
        pltpu.SemaphoreType.DMA,
        pltpu.SemaphoreType.DMA,
    ],
)
def _sc_gather(words_hbm, ctx_hbm, neg_hbm, wtab_hbm, ctab_hbm,
               out_w, out_c, out_n,
               idx_w, idx_c, idx_n, buf_a0, buf_a1, buf_b0, buf_b1,
               gsem_a, gsem_b, ssem_a, ssem_b):
    wid = lax.axis_index("s") * NC + lax.axis_index("c")

    pltpu.sync_copy(words_hbm.at[wid], idx_w)
    pltpu.sync_copy(ctx_hbm.at[wid], idx_c)
    pltpu.sync_copy(neg_hbm.at[wid], idx_n)

    def phase(tab, idx_v, out, base, nchunks):
        def g_start(j, buf, sem):
            pltpu.async_copy(tab.at[idx_v.at[j]], buf, sem)

        def g_wait(j, buf, sem):
            pltpu.make_async_copy(tab.at[idx_v.at[j]], buf, sem).wait()

        def s_start(j, buf, sem):
            pltpu.async_copy(buf, out.at[pl.ds(base + j * CH, CH)], sem)

        def s_wait(j, buf, sem):
            pltpu.make_async_copy(
                buf, out.at[pl.ds(base + j * CH, CH)], sem).wait()

        g_start(0, buf_a0, gsem_a)
        g_start(1, buf_a1, gsem_a)
        g_start(2, buf_b0, gsem_b)
        g_start(3, buf_b1, gsem_b)

        def body(m, carry):
            c = m * 4
            g_wait(c, buf_a0, gsem_a)
            g_wait(c + 1, buf_a1, gsem_a)
            s_start(c, buf_a0, ssem_a)
            s_start(c + 1, buf_a1, ssem_a)
            s_wait(c, buf_a0, ssem_a)
            s_wait(c + 1, buf_a1, ssem_a)
            g_start(c + 4, buf_a0, gsem_a)
            g_start(c + 5, buf_a1, gsem_a)
            g_wait(c + 2, buf_b0, gsem_b)
            g_wait(c + 3, buf_b1, gsem_b)
            s_start(c + 2, buf_b0, ssem_b)
            s_start(c + 3, buf_b1, ssem_b)
            s_wait(c + 2, buf_b0, ssem_b)
            s_wait(c + 3, buf_b1, ssem_b)
            g_start(c + 6, buf_b0, gsem_b)
            g_start(c + 7, buf_b1, gsem_b)
            return carry
        lax.fori_loop(0, nchunks // 4 - 1, body, 0)

        c = nchunks - 4
        g_wait(c, buf_a0, gsem_a)
        g_wait(c + 1, buf_a1, gsem_a)
        s_start(c, buf_a0, ssem_a)
        s_start(c + 1, buf_a1, ssem_a)
        g_wait(c + 2, buf_b0, gsem_b)
        g_wait(c + 3, buf_b1, gsem_b)
        s_start(c + 2, buf_b0, ssem_b)
        s_start(c + 3, buf_b1, ssem_b)
        s_wait(c, buf_a0, ssem_a)
        s_wait(c + 1, buf_a1, ssem_a)
        s_wait(c + 2, buf_b0, ssem_b)
        s_wait(c + 3, buf_b1, ssem_b)

    phase(wtab_hbm, idx_w, out_w, wid * W_PER, W_CHUNKS)
    phase(ctab_hbm, idx_c, out_c, wid * W_PER, W_CHUNKS)
    phase(ctab_hbm, idx_n, out_n, wid * N_PER, N_CHUNKS)


def kernel(words, contexts, neg_contexts, w_embeddings, c_embeddings):
    words3 = words.astype(jnp.int32).reshape(NW, W_CHUNKS, CH)
    ctx3 = contexts.astype(jnp.int32).reshape(NW, W_CHUNKS, CH)
    neg3 = neg_contexts.astype(jnp.int32).T.reshape(NW, N_CHUNKS, CH)
    out_w, out_c, out_nf = _sc_gather(words3, ctx3, neg3,
                                      w_embeddings, c_embeddings)
    out_n = out_nf.reshape(NEG, BATCH, DIM).transpose(1, 0, 2)
    return (out_w, out_c, out_n)
